# baseline (speedup 1.0000x reference)
import jax
import jax.numpy as jnp
from jax import lax
from jax.experimental import pallas as pl
from jax.experimental.pallas import tpu as pltpu
from jax.experimental.pallas import tpu_sc as plsc

N = 10000
E = 320000
D = 128

NC = 2
NS = 16
NW = NC * NS

B = 125
CH = 80
G = 16
NG = CH // G

NWRITERS = 10
ROWS_PER_WRITER = N // NWRITERS


def _sc_aggregate_body(x_hbm, src_hbm, dst_hbm, zeros_hbm, out_hbm,
                       src_v, dst_v, rows_a, rows_b, acc,
                       gsem_a, gsem_b, ssem_a, ssem_b, isem_a, isem_b):
    cid = lax.axis_index("c")
    sid = lax.axis_index("s")
    wid = cid * NS + sid

    row0 = sid * ROWS_PER_WRITER

    @pl.when(sid < NWRITERS)
    def _zero():
        pltpu.sync_copy(zeros_hbm.at[pl.ds(row0, ROWS_PER_WRITER)],
                        acc.at[pl.ds(row0, ROWS_PER_WRITER)])
    plsc.subcore_barrier()

    rows = (rows_a, rows_b)
    gsems = (gsem_a, gsem_b)
    ssems = (ssem_a, ssem_b)

    def stage_idx(g, slot):
        pltpu.async_copy(src_hbm.at[wid, pl.ds(g * G, G)], src_v.at[slot],
                         isem_a)
        pltpu.async_copy(dst_hbm.at[wid, pl.ds(g * G, G)], dst_v.at[slot],
                         isem_b)

    def wait_idx(g, slot):
        pltpu.make_async_copy(src_hbm.at[wid, pl.ds(g * G, G)],
                              src_v.at[slot], isem_a).wait()
        pltpu.make_async_copy(dst_hbm.at[wid, pl.ds(g * G, G)],
                              dst_v.at[slot], isem_b).wait()

    def gather_start(slot, j, buf):
        pltpu.async_copy(x_hbm.at[src_v.at[slot, j]], rows[buf], gsems[buf])

    def gather_wait(slot, j, buf):
        pltpu.make_async_copy(x_hbm.at[src_v.at[slot, j]], rows[buf],
                              gsems[buf]).wait()

    def scatter_start(slot, j, buf):
        pltpu.async_copy(rows[buf], acc.at[dst_v.at[slot, j]],
                         ssems[buf], add=True)

    def scatter_wait(slot, j, buf):
        pltpu.make_async_copy(rows[buf], acc.at[dst_v.at[slot, j]],
                              ssems[buf]).wait()

    stage_idx(0, 0)
    wait_idx(0, 0)
    gather_start(0, 0, 0)

    def group(g, _):
        slot = g % 2
        oslot = 1 - slot

        for j in range(G):
            par = (j + 1) % 2
            if j == 0:
                @pl.when(g >= 1)
                def _drain_prev_group():
                    scatter_wait(oslot, G - 1, par)
                gather_start(slot, 1, par)
            elif j == 1:
                @pl.when(g + 1 < NG)
                def _stage_next():
                    stage_idx(g + 1, oslot)
                scatter_wait(slot, 0, par)
                gather_start(slot, 2, par)
            elif j + 1 < G:
                scatter_wait(slot, j - 1, par)
                gather_start(slot, j + 1, par)
            else:
                @pl.when(g + 1 < NG)
                def _prefetch_group():
                    scatter_wait(slot, G - 2, par)
                    wait_idx(g + 1, oslot)
                    gather_start(oslot, 0, par)
            gather_wait(slot, j, j % 2)
            scatter_start(slot, j, j % 2)
        return ()

    lax.fori_loop(0, NG, group, (), unroll=False)

    scatter_wait((NG - 1) % 2, G - 2, (G - 2) % 2)
    scatter_wait((NG - 1) % 2, G - 1, (G - 1) % 2)

    plsc.subcore_barrier()

    @pl.when(sid < NWRITERS)
    def _publish():
        pltpu.sync_copy(acc.at[pl.ds(row0, ROWS_PER_WRITER)],
                        out_hbm.at[cid, pl.ds(row0, ROWS_PER_WRITER)])


@jax.jit
def _sc_aggregate(x, src_p, dst_p, zeros_full):
    mesh = plsc.VectorSubcoreMesh(core_axis_name="c", subcore_axis_name="s")
    return pl.kernel(
        _sc_aggregate_body,
        out_type=jax.ShapeDtypeStruct((NC, N, D), jnp.float32),
        mesh=mesh,
        scratch_types=[
            pltpu.VMEM((2, G, B), jnp.int32),
            pltpu.VMEM((2, G, B), jnp.int32),
            pltpu.VMEM((B, D), jnp.float32),
            pltpu.VMEM((B, D), jnp.float32),
            pltpu.VMEM_SHARED((N, D), jnp.float32),
            pltpu.SemaphoreType.DMA,
            pltpu.SemaphoreType.DMA,
            pltpu.SemaphoreType.DMA,
            pltpu.SemaphoreType.DMA,
            pltpu.SemaphoreType.DMA,
            pltpu.SemaphoreType.DMA,
        ],
    )(x, src_p, dst_p, zeros_full)


def _tc_combine_body(p_ref, w_ref, b_ref, o_ref):
    s = p_ref[0] + p_ref[1]
    y = jnp.dot(s, w_ref[...], preferred_element_type=jnp.float32,
                precision=jax.lax.Precision.HIGHEST)
    o_ref[...] = jnp.maximum(y + b_ref[...], 0.0)


BM = 400


@jax.jit
def _tc_combine(partials, weight, bias2d):
    return pl.pallas_call(
        _tc_combine_body,
        grid=(N // BM,),
        in_specs=[
            pl.BlockSpec((NC, BM, D), lambda i: (0, i, 0)),
            pl.BlockSpec((D, D), lambda i: (0, 0)),
            pl.BlockSpec((1, D), lambda i: (0, 0)),
        ],
        out_specs=pl.BlockSpec((BM, D), lambda i: (i, 0)),
        out_shape=jax.ShapeDtypeStruct((N, D), jnp.float32),
    )(partials, weight, bias2d)


def kernel(adj, x, weight, bias):
    dst_p = adj[0].reshape(NW, CH, B)
    src_p = adj[1].reshape(NW, CH, B)
    zeros_full = jnp.zeros((N, D), jnp.float32)

    partials = _sc_aggregate(x, src_p, dst_p, zeros_full)
    return _tc_combine(partials, weight, bias.reshape(1, D))


# speedup vs baseline: 1.0191x; 1.0191x over previous
import jax
import jax.numpy as jnp
from jax import lax
from jax.experimental import pallas as pl
from jax.experimental.pallas import tpu as pltpu
from jax.experimental.pallas import tpu_sc as plsc

N = 10000
E = 320000
D = 128

NC = 2
NS = 16
NW = NC * NS

B = 128
CH = 80
E_PAD = NW * CH * B
G = 16
NG = CH // G

N_PAD = 10112
ROWS_PER_WRITER = N_PAD // NS


def _sc_aggregate_body(x_hbm, src_hbm, dst_hbm, zeros_hbm, out_hbm,
                       src_v, dst_v, rows_a, rows_b, acc,
                       gsem_a, gsem_b, ssem_a, ssem_b, isem_a, isem_b):
    cid = lax.axis_index("c")
    sid = lax.axis_index("s")
    wid = cid * NS + sid

    row0 = sid * ROWS_PER_WRITER

    pltpu.sync_copy(zeros_hbm.at[pl.ds(row0, ROWS_PER_WRITER)],
                    acc.at[pl.ds(row0, ROWS_PER_WRITER)])
    plsc.subcore_barrier()

    rows = (rows_a, rows_b)
    gsems = (gsem_a, gsem_b)
    ssems = (ssem_a, ssem_b)

    def stage_idx(g, slot):
        pltpu.async_copy(src_hbm.at[wid, pl.ds(g * G, G)], src_v.at[slot],
                         isem_a)
        pltpu.async_copy(dst_hbm.at[wid, pl.ds(g * G, G)], dst_v.at[slot],
                         isem_b)

    def wait_idx(g, slot):
        pltpu.make_async_copy(src_hbm.at[wid, pl.ds(g * G, G)],
                              src_v.at[slot], isem_a).wait()
        pltpu.make_async_copy(dst_hbm.at[wid, pl.ds(g * G, G)],
                              dst_v.at[slot], isem_b).wait()

    def gather_start(slot, j, buf):
        pltpu.async_copy(x_hbm.at[src_v.at[slot, j]], rows[buf], gsems[buf])

    def gather_wait(slot, j, buf):
        pltpu.make_async_copy(x_hbm.at[src_v.at[slot, j]], rows[buf],
                              gsems[buf]).wait()

    def scatter_start(slot, j, buf):
        pltpu.async_copy(rows[buf], acc.at[dst_v.at[slot, j]],
                         ssems[buf], add=True)

    def scatter_wait(slot, j, buf):
        pltpu.make_async_copy(rows[buf], acc.at[dst_v.at[slot, j]],
                              ssems[buf]).wait()

    stage_idx(0, 0)
    wait_idx(0, 0)
    gather_start(0, 0, 0)

    def group(g, _):
        slot = g % 2
        oslot = 1 - slot

        for j in range(G):
            par = (j + 1) % 2
            if j == 0:
                @pl.when(g >= 1)
                def _drain_prev_group():
                    scatter_wait(oslot, G - 1, par)
                gather_start(slot, 1, par)
            elif j == 1:
                @pl.when(g + 1 < NG)
                def _stage_next():
                    stage_idx(g + 1, oslot)
                scatter_wait(slot, 0, par)
                gather_start(slot, 2, par)
            elif j + 1 < G:
                scatter_wait(slot, j - 1, par)
                gather_start(slot, j + 1, par)
            else:
                @pl.when(g + 1 < NG)
                def _prefetch_group():
                    scatter_wait(slot, G - 2, par)
                    wait_idx(g + 1, oslot)
                    gather_start(oslot, 0, par)
            gather_wait(slot, j, j % 2)
            scatter_start(slot, j, j % 2)
        return ()

    lax.fori_loop(0, NG, group, (), unroll=False)

    scatter_wait((NG - 1) % 2, G - 2, (G - 2) % 2)
    scatter_wait((NG - 1) % 2, G - 1, (G - 1) % 2)

    plsc.subcore_barrier()
    pltpu.sync_copy(acc.at[pl.ds(row0, ROWS_PER_WRITER)],
                    out_hbm.at[cid, pl.ds(row0, ROWS_PER_WRITER)])


@jax.jit
def _sc_aggregate(x, src_p, dst_p, zeros_full):
    mesh = plsc.VectorSubcoreMesh(core_axis_name="c", subcore_axis_name="s")
    return pl.kernel(
        _sc_aggregate_body,
        out_type=jax.ShapeDtypeStruct((NC, N_PAD, D), jnp.float32),
        mesh=mesh,
        scratch_types=[
            pltpu.VMEM((2, G, B), jnp.int32),
            pltpu.VMEM((2, G, B), jnp.int32),
            pltpu.VMEM((B, D), jnp.float32),
            pltpu.VMEM((B, D), jnp.float32),
            pltpu.VMEM_SHARED((N_PAD, D), jnp.float32),
            pltpu.SemaphoreType.DMA,
            pltpu.SemaphoreType.DMA,
            pltpu.SemaphoreType.DMA,
            pltpu.SemaphoreType.DMA,
            pltpu.SemaphoreType.DMA,
            pltpu.SemaphoreType.DMA,
        ],
    )(x, src_p, dst_p, zeros_full)


def _tc_combine_body(p_ref, w_ref, b_ref, o_ref):
    s = p_ref[0] + p_ref[1]
    y = jnp.dot(s, w_ref[...], preferred_element_type=jnp.float32,
                precision=jax.lax.Precision.HIGHEST)
    o_ref[...] = jnp.maximum(y + b_ref[...], 0.0)


BM = 400


@jax.jit
def _tc_combine(partials, weight, bias2d):
    return pl.pallas_call(
        _tc_combine_body,
        grid=(N // BM,),
        in_specs=[
            pl.BlockSpec((NC, BM, D), lambda i: (0, i, 0)),
            pl.BlockSpec((D, D), lambda i: (0, 0)),
            pl.BlockSpec((1, D), lambda i: (0, 0)),
        ],
        out_specs=pl.BlockSpec((BM, D), lambda i: (i, 0)),
        out_shape=jax.ShapeDtypeStruct((N, D), jnp.float32),
    )(partials, weight, bias2d)


def kernel(adj, x, weight, bias):
    dst = adj[0]
    src = adj[1]
    pad = E_PAD - E
    pad_iota = jnp.arange(pad, dtype=jnp.int32)
    src_p = jnp.concatenate([src, pad_iota % N]).reshape(NW, CH, B)
    dst_p = jnp.concatenate([dst, N + pad_iota % (N_PAD - N)]).reshape(NW, CH, B)
    zeros_full = jnp.zeros((N_PAD, D), jnp.float32)

    partials = _sc_aggregate(x, src_p, dst_p, zeros_full)
    return _tc_combine(partials, weight, bias.reshape(1, D))


# speedup vs baseline: 1.1267x; 1.1056x over previous
import jax
import jax.numpy as jnp
from jax import lax
from jax.experimental import pallas as pl
from jax.experimental.pallas import tpu as pltpu
from jax.experimental.pallas import tpu_sc as plsc

N = 10000
E = 320000
D = 128

NC = 2
NS = 16
NW = NC * NS

B = 128
CH = 80
E_PAD = NW * CH * B
G = 16
NG = CH // G

N_PAD = 10112
ROWS_PER_WRITER = N_PAD // NS


def _sc_aggregate_body(x_hbm, src_hbm, dst_hbm, out_hbm,
                       src_v, dst_v, rows_a, rows_b, acc,
                       gsem_a, gsem_b, ssem_a, ssem_b, isem_a, isem_b):
    cid = lax.axis_index("c")
    sid = lax.axis_index("s")
    wid = cid * NS + sid

    row0 = sid * ROWS_PER_WRITER

    zv = jnp.zeros((16,), jnp.float32)

    def _zero_row(r, _):
        for k in range(D // 16):
            rows_a[r, pl.ds(k * 16, 16)] = zv
        return ()

    lax.fori_loop(0, B, _zero_row, (), unroll=False)
    nfull = ROWS_PER_WRITER // B
    rem = ROWS_PER_WRITER - nfull * B
    for i in range(nfull):
        pltpu.sync_copy(rows_a, acc.at[pl.ds(row0 + i * B, B)])
    if rem:
        pltpu.sync_copy(rows_a.at[pl.ds(0, rem)],
                        acc.at[pl.ds(row0 + nfull * B, rem)])
    plsc.subcore_barrier()

    rows = (rows_a, rows_b)
    gsems = (gsem_a, gsem_b)
    ssems = (ssem_a, ssem_b)

    def stage_idx(g, slot):
        pltpu.async_copy(src_hbm.at[wid, pl.ds(g * G, G)], src_v.at[slot],
                         isem_a)
        pltpu.async_copy(dst_hbm.at[wid, pl.ds(g * G, G)], dst_v.at[slot],
                         isem_b)

    def wait_idx(g, slot):
        pltpu.make_async_copy(src_hbm.at[wid, pl.ds(g * G, G)],
                              src_v.at[slot], isem_a).wait()
        pltpu.make_async_copy(dst_hbm.at[wid, pl.ds(g * G, G)],
                              dst_v.at[slot], isem_b).wait()

    def gather_start(slot, j, buf):
        pltpu.async_copy(x_hbm.at[src_v.at[slot, j]], rows[buf], gsems[buf])

    def gather_wait(slot, j, buf):
        pltpu.make_async_copy(x_hbm.at[src_v.at[slot, j]], rows[buf],
                              gsems[buf]).wait()

    def scatter_start(slot, j, buf):
        pltpu.async_copy(rows[buf], acc.at[dst_v.at[slot, j]],
                         ssems[buf], add=True)

    def scatter_wait(slot, j, buf):
        pltpu.make_async_copy(rows[buf], acc.at[dst_v.at[slot, j]],
                              ssems[buf]).wait()

    stage_idx(0, 0)
    wait_idx(0, 0)
    gather_start(0, 0, 0)

    def group(g, _):
        slot = g % 2
        oslot = 1 - slot

        for j in range(G):
            par = (j + 1) % 2
            if j == 0:
                @pl.when(g >= 1)
                def _drain_prev_group():
                    scatter_wait(oslot, G - 1, par)
                gather_start(slot, 1, par)
            elif j == 1:
                @pl.when(g + 1 < NG)
                def _stage_next():
                    stage_idx(g + 1, oslot)
                scatter_wait(slot, 0, par)
                gather_start(slot, 2, par)
            elif j + 1 < G:
                scatter_wait(slot, j - 1, par)
                gather_start(slot, j + 1, par)
            else:
                @pl.when(g + 1 < NG)
                def _prefetch_group():
                    scatter_wait(slot, G - 2, par)
                    wait_idx(g + 1, oslot)
                    gather_start(oslot, 0, par)
            gather_wait(slot, j, j % 2)
            scatter_start(slot, j, j % 2)
        return ()

    lax.fori_loop(0, NG, group, (), unroll=False)

    scatter_wait((NG - 1) % 2, G - 2, (G - 2) % 2)
    scatter_wait((NG - 1) % 2, G - 1, (G - 1) % 2)

    plsc.subcore_barrier()
    pltpu.sync_copy(acc.at[pl.ds(row0, ROWS_PER_WRITER)],
                    out_hbm.at[cid, pl.ds(row0, ROWS_PER_WRITER)])


@jax.jit
def _sc_aggregate(x, src_p, dst_p):
    mesh = plsc.VectorSubcoreMesh(core_axis_name="c", subcore_axis_name="s")
    return pl.kernel(
        _sc_aggregate_body,
        out_type=jax.ShapeDtypeStruct((NC, N_PAD, D), jnp.float32),
        mesh=mesh,
        scratch_types=[
            pltpu.VMEM((2, G, B), jnp.int32),
            pltpu.VMEM((2, G, B), jnp.int32),
            pltpu.VMEM((B, D), jnp.float32),
            pltpu.VMEM((B, D), jnp.float32),
            pltpu.VMEM_SHARED((N_PAD, D), jnp.float32),
            pltpu.SemaphoreType.DMA,
            pltpu.SemaphoreType.DMA,
            pltpu.SemaphoreType.DMA,
            pltpu.SemaphoreType.DMA,
            pltpu.SemaphoreType.DMA,
            pltpu.SemaphoreType.DMA,
        ],
    )(x, src_p, dst_p)


def _tc_combine_body(p_ref, w_ref, b_ref, o_ref):
    s = p_ref[0] + p_ref[1]
    y = jnp.dot(s, w_ref[...], preferred_element_type=jnp.float32,
                precision=jax.lax.Precision.HIGHEST)
    o_ref[...] = jnp.maximum(y + b_ref[...], 0.0)


BM = 2000


@jax.jit
def _tc_combine(partials, weight, bias2d):
    return pl.pallas_call(
        _tc_combine_body,
        grid=(N // BM,),
        in_specs=[
            pl.BlockSpec((NC, BM, D), lambda i: (0, i, 0)),
            pl.BlockSpec((D, D), lambda i: (0, 0)),
            pl.BlockSpec((1, D), lambda i: (0, 0)),
        ],
        out_specs=pl.BlockSpec((BM, D), lambda i: (i, 0)),
        out_shape=jax.ShapeDtypeStruct((N, D), jnp.float32),
    )(partials, weight, bias2d)


def kernel(adj, x, weight, bias):
    dst = adj[0]
    src = adj[1]
    pad = E_PAD - E
    pad_iota = jnp.arange(pad, dtype=jnp.int32)
    src_p = jnp.concatenate([src, pad_iota % N]).reshape(NW, CH, B)
    dst_p = jnp.concatenate([dst, N + pad_iota % (N_PAD - N)]).reshape(NW, CH, B)

    partials = _sc_aggregate(x, src_p, dst_p)
    return _tc_combine(partials, weight, bias.reshape(1, D))


# speedup vs baseline: 1.1559x; 1.0260x over previous
import jax
import jax.numpy as jnp
from jax import lax
from jax.experimental import pallas as pl
from jax.experimental.pallas import tpu as pltpu
from jax.experimental.pallas import tpu_sc as plsc

N = 10000
E = 320000
D = 128

NC = 2
NS = 16
NW = NC * NS

B = 128
CH = 80
E_PAD = NW * CH * B
G = 16
NG = CH // G

N_PAD = 10112
ROWS_PER_WRITER = N_PAD // NS


def _sc_aggregate_body(x_hbm, src_hbm, dst_hbm, out_hbm,
                       src_v, dst_v, rows_a, rows_b, acc,
                       gsem_a, gsem_b, ssem_a, ssem_b, isem_a, isem_b):
    cid = lax.axis_index("c")
    sid = lax.axis_index("s")
    wid = cid * NS + sid

    row0 = sid * ROWS_PER_WRITER

    zv = jnp.zeros((16,), jnp.float32)

    def _zero_row(r, _):
        for k in range(D // 16):
            rows_a[r, pl.ds(k * 16, 16)] = zv
        return ()

    lax.fori_loop(0, B, _zero_row, (), unroll=False)
    nfull = ROWS_PER_WRITER // B
    rem = ROWS_PER_WRITER - nfull * B
    for i in range(nfull):
        pltpu.sync_copy(rows_a, acc.at[pl.ds(row0 + i * B, B)])
    if rem:
        pltpu.sync_copy(rows_a.at[pl.ds(0, rem)],
                        acc.at[pl.ds(row0 + nfull * B, rem)])
    plsc.subcore_barrier()

    rows = (rows_a, rows_b)
    gsems = (gsem_a, gsem_b)
    ssems = (ssem_a, ssem_b)

    def stage_idx(g, slot):
        pltpu.async_copy(src_hbm.at[wid, pl.ds(g * G, G)], src_v.at[slot],
                         isem_a)
        pltpu.async_copy(dst_hbm.at[wid, pl.ds(g * G, G)], dst_v.at[slot],
                         isem_b)

    def wait_idx(g, slot):
        pltpu.make_async_copy(src_hbm.at[wid, pl.ds(g * G, G)],
                              src_v.at[slot], isem_a).wait()
        pltpu.make_async_copy(dst_hbm.at[wid, pl.ds(g * G, G)],
                              dst_v.at[slot], isem_b).wait()

    def gather_start(slot, j, buf):
        pltpu.async_copy(x_hbm.at[src_v.at[slot, j]], rows[buf], gsems[buf])

    def gather_wait(slot, j, buf):
        pltpu.make_async_copy(x_hbm.at[src_v.at[slot, j]], rows[buf],
                              gsems[buf]).wait()

    def scatter_start(slot, j, buf):
        pltpu.async_copy(rows[buf], acc.at[dst_v.at[slot, j]],
                         ssems[buf], add=True)

    def scatter_wait(slot, j, buf):
        pltpu.make_async_copy(rows[buf], acc.at[dst_v.at[slot, j]],
                              ssems[buf]).wait()

    stage_idx(0, 0)
    wait_idx(0, 0)
    gather_start(0, 0, 0)

    def group(g, _):
        slot = g % 2
        oslot = 1 - slot

        for j in range(G):
            par = (j + 1) % 2
            if j == 0:
                @pl.when(g >= 1)
                def _drain_prev_group():
                    scatter_wait(oslot, G - 1, par)
                gather_start(slot, 1, par)
            elif j == 1:
                @pl.when(g + 1 < NG)
                def _stage_next():
                    stage_idx(g + 1, oslot)
                scatter_wait(slot, 0, par)
                gather_start(slot, 2, par)
            elif j + 1 < G:
                scatter_wait(slot, j - 1, par)
                gather_start(slot, j + 1, par)
            else:
                @pl.when(g + 1 < NG)
                def _prefetch_group():
                    scatter_wait(slot, G - 2, par)
                    wait_idx(g + 1, oslot)
                    gather_start(oslot, 0, par)
            gather_wait(slot, j, j % 2)
            scatter_start(slot, j, j % 2)
        return ()

    lax.fori_loop(0, NG, group, (), unroll=False)

    scatter_wait((NG - 1) % 2, G - 2, (G - 2) % 2)
    scatter_wait((NG - 1) % 2, G - 1, (G - 1) % 2)

    plsc.subcore_barrier()
    pltpu.sync_copy(acc.at[pl.ds(row0, ROWS_PER_WRITER)],
                    out_hbm.at[cid, pl.ds(row0, ROWS_PER_WRITER)])


@jax.jit
def _sc_aggregate(x, src_p, dst_p):
    mesh = plsc.VectorSubcoreMesh(core_axis_name="c", subcore_axis_name="s")
    return pl.kernel(
        _sc_aggregate_body,
        out_type=jax.ShapeDtypeStruct((NC, N_PAD, D), jnp.float32),
        mesh=mesh,
        scratch_types=[
            pltpu.VMEM((2, G, B), jnp.int32),
            pltpu.VMEM((2, G, B), jnp.int32),
            pltpu.VMEM((B, D), jnp.float32),
            pltpu.VMEM((B, D), jnp.float32),
            pltpu.VMEM_SHARED((N_PAD, D), jnp.float32),
            pltpu.SemaphoreType.DMA,
            pltpu.SemaphoreType.DMA,
            pltpu.SemaphoreType.DMA,
            pltpu.SemaphoreType.DMA,
            pltpu.SemaphoreType.DMA,
            pltpu.SemaphoreType.DMA,
        ],
    )(x, src_p, dst_p)


def _tc_combine_body(p_ref, w_ref, b_ref, o_ref):
    s = p_ref[0] + p_ref[1]
    y = jnp.dot(s, w_ref[...], preferred_element_type=jnp.float32)
    o_ref[...] = jnp.maximum(y + b_ref[...], 0.0)


BM = 5000


@jax.jit
def _tc_combine(partials, weight, bias2d):
    return pl.pallas_call(
        _tc_combine_body,
        grid=(N // BM,),
        in_specs=[
            pl.BlockSpec((NC, BM, D), lambda i: (0, i, 0)),
            pl.BlockSpec((D, D), lambda i: (0, 0)),
            pl.BlockSpec((1, D), lambda i: (0, 0)),
        ],
        out_specs=pl.BlockSpec((BM, D), lambda i: (i, 0)),
        out_shape=jax.ShapeDtypeStruct((N, D), jnp.float32),
    )(partials, weight, bias2d)


def kernel(adj, x, weight, bias):
    dst = adj[0]
    src = adj[1]
    pad = E_PAD - E
    pad_iota = jnp.arange(pad, dtype=jnp.int32)
    src_p = jnp.concatenate([src, pad_iota % N]).reshape(NW, CH, B)
    dst_p = jnp.concatenate([dst, N + pad_iota % (N_PAD - N)]).reshape(NW, CH, B)

    partials = _sc_aggregate(x, src_p, dst_p)
    return _tc_combine(partials, weight, bias.reshape(1, D))
